# Initial kernel scaffold; baseline (speedup 1.0000x reference)
import jax
import jax.numpy as jnp
from jax.experimental import pallas as pl


def kernel(boxes, scores):
    raise NotImplementedError("write your pallas kernel here")


# speedup vs baseline: 154.0997x; 154.0997x over previous
import functools

import jax
import jax.numpy as jnp
from jax import lax
from jax.experimental import pallas as pl
from jax.experimental.pallas import tpu as pltpu

N_BOXES = 20000
IOU_T = 0.5
B = 1024


def _nms_body(x1r, y1r, x2r, y2r, eye, keep_out, x2p, nb):
    j = pl.program_id(0)

    @pl.when(j == 0)
    def _init():
        x2p[...] = x2r[...]

    def row(ref, i):
        return ref[pl.ds(i, 1)].reshape(1, B)

    def to_col(v_row):
        return lax.dot_general(eye[...], v_row, (((1,), (1,)), ((), ())),
                               preferred_element_type=jnp.float32)

    x1j = to_col(row(x1r, j))
    y1j = to_col(row(y1r, j))
    x2j = to_col(row(x2r, j))
    y2j = to_col(row(y2r, j))
    aj = jnp.maximum(x2j - x1j, 0.0) * jnp.maximum(y2j - y1j, 0.0)

    def pair_sup(x1i, y1i, x2i, y2i):
        ai = jnp.maximum(x2i - x1i, 0.0) * jnp.maximum(y2i - y1i, 0.0)
        w = jnp.maximum(jnp.minimum(x2i, x2j) - jnp.maximum(x1i, x1j), 0.0)
        h = jnp.maximum(jnp.minimum(y2i, y2j) - jnp.maximum(y1i, y1j), 0.0)
        inter = w * h
        denom = ai + aj - inter + 1e-12
        return inter > IOU_T * denom

    def cross_body(i, sup):
        m = pair_sup(row(x1r, i), row(y1r, i), row(x2p, i), row(y2r, i))
        return jnp.maximum(
            sup, jnp.max(m.astype(jnp.float32), axis=1, keepdims=True))

    sup = lax.fori_loop(0, j, cross_body,
                        jnp.zeros((B, 1), dtype=jnp.float32))

    tri = (lax.broadcasted_iota(jnp.int32, (B, B), 1)
           < lax.broadcasted_iota(jnp.int32, (B, B), 0))
    m_intra = (pair_sup(row(x1r, j), row(y1r, j), row(x2r, j), row(y2r, j))
               & tri).astype(jnp.float32)

    k0 = 1.0 - sup

    def fp_cond(st):
        _, changed, it = st
        return jnp.logical_and(changed > 0, it < B)

    def fp_body(st):
        k, _, it = st
        scol = lax.dot_general(m_intra, k, (((1,), (0,)), ((), ())),
                               preferred_element_type=jnp.float32)
        kn = jnp.where(scol > 0.0, 0.0, k0)
        changed = jnp.sum((kn != k).astype(jnp.int32))
        return kn, changed, it + 1

    k, _, _ = lax.while_loop(
        fp_cond, fp_body, (k0, jnp.int32(1), jnp.int32(0)))

    k_row = lax.dot_general(k, eye[...], (((0,), (0,)), ((), ())),
                            preferred_element_type=jnp.float32)
    keep_out[pl.ds(j, 1)] = k_row.reshape(1, 1, B)
    x2p[pl.ds(j, 1)] = jnp.where(k_row > 0.0, row(x2r, j), -1e9).reshape(
        1, 1, B)


def _nms_keep(bp):
    npad, _ = bp.shape
    nb = npad // B
    x1 = bp[:, 0].reshape(nb, 1, B)
    y1 = bp[:, 1].reshape(nb, 1, B)
    x2 = bp[:, 2].reshape(nb, 1, B)
    y2 = bp[:, 3].reshape(nb, 1, B)
    eye = jnp.eye(B, dtype=jnp.float32)
    full_r = pl.BlockSpec((nb, 1, B), lambda j: (0, 0, 0))
    full_e = pl.BlockSpec((B, B), lambda j: (0, 0))
    keep = pl.pallas_call(
        functools.partial(_nms_body, nb=nb),
        grid=(nb,),
        in_specs=[full_r, full_r, full_r, full_r, full_e],
        out_specs=full_r,
        out_shape=jax.ShapeDtypeStruct((nb, 1, B), jnp.float32),
        scratch_shapes=[pltpu.VMEM((nb, 1, B), jnp.float32)],
    )(x1, y1, x2, y2, eye)
    return keep.reshape(npad)


def kernel(boxes, scores):
    n = boxes.shape[0]
    order = jnp.argsort(-scores)
    b = jnp.take(boxes, order, axis=0)
    s = jnp.take(scores, order, axis=0)
    npad = ((n + B - 1) // B) * B
    pad = jnp.tile(jnp.array([[0.0, 0.0, -1.0, -1.0]], jnp.float32),
                   (npad - n, 1))
    bp = jnp.concatenate([b, pad], axis=0)
    keep = _nms_keep(bp)[:n]
    out = jnp.concatenate([b * keep[:, None], (s * keep)[:, None]], axis=1)
    return out
